# baseline (speedup 1.0000x reference)
import functools

import jax
import jax.numpy as jnp
from jax import lax
from jax.experimental import pallas as pl
from jax.experimental.pallas import tpu as pltpu
from jax.experimental.pallas import tpu_sc as plsc

B = 4096
F = 26
V = 100000
D = 32
DENSE = 13
EPS = 1e-5
TB = 2048
FH = F // 2


def _make_sc_gather(f_off):
    info = plsc.get_sparse_core_info()
    nc, ns, nl = info.num_cores, info.num_subcores, info.num_lanes
    nw = nc * ns
    rows_per_w = (FH * D) // nw

    mesh = plsc.VectorSubcoreMesh(core_axis_name="c", subcore_axis_name="s")

    @functools.partial(
        pl.kernel,
        out_type=[
            jax.ShapeDtypeStruct((FH * D, B), jnp.float32),
            jax.ShapeDtypeStruct((FH, B), jnp.float32),
        ],
        mesh=mesh,
        compiler_params=pltpu.CompilerParams(needs_layout_passes=False),
        scratch_types=[
            pltpu.VMEM((V,), jnp.float32),
            pltpu.VMEM((B,), jnp.int32),
            pltpu.VMEM((2, B), jnp.float32),
            pltpu.SemaphoreType.DMA,
        ],
    )
    def sc_gather(idx_hbm, etab_hbm, ltab_hbm, emb_out, lin_out,
                  tab_v, idx_v, out_v, sem_o):
        wid = lax.axis_index("s") * nc + lax.axis_index("c")
        r0 = wid * rows_per_w

        def gather_staged(jb):
            def chunk(k, _):
                base = pl.multiple_of(k * 128, 128)
                for u in range(8):
                    off = base + u * nl
                    iv = idx_v[pl.ds(off, nl)]
                    out_v[jb, pl.ds(off, nl)] = plsc.load_gather(tab_v, [iv])
                return _

            lax.fori_loop(0, B // 128, chunk, None)

        def row_body(j, _):
            r = r0 + j
            f = f_off + r // D
            d = r % D

            @pl.when((j == 0) | (d == 0))
            def _():
                pltpu.sync_copy(idx_hbm.at[f], idx_v)

            pltpu.sync_copy(etab_hbm.at[f, d], tab_v)
            jb = j % 2

            @pl.when(j >= 2)
            def _():
                pltpu.make_async_copy(out_v.at[jb], emb_out.at[r], sem_o).wait()

            gather_staged(jb)
            pltpu.make_async_copy(out_v.at[jb], emb_out.at[r], sem_o).start()
            return _

        lax.fori_loop(0, rows_per_w, row_body, None)
        pltpu.make_async_copy(out_v.at[0], emb_out.at[r0], sem_o).wait()
        pltpu.make_async_copy(out_v.at[0], emb_out.at[r0], sem_o).wait()

        @pl.when(wid < FH)
        def _():
            pltpu.sync_copy(idx_hbm.at[f_off + wid], idx_v)
            pltpu.sync_copy(ltab_hbm.at[f_off + wid], tab_v)
            gather_staged(0)
            pltpu.sync_copy(out_v.at[0], lin_out.at[wid])

    return sc_gather


def _t_dot(w_ref, x):
    return lax.dot_general(w_ref[...], x, (((0,), (0,)), ((), ())),
                           preferred_element_type=jnp.float32)


def _stage1a_body(dense_ref, emb_ref, lin_ref, w1d_ref, w1e_ref,
                  wd_ref, bd_ref, a_ref, h1_ref, s_ref, aux_ref):
    emb = emb_ref[...]
    dense = dense_ref[...]
    h1_ref[...] = _t_dot(w1d_ref, dense) + _t_dot(w1e_ref, emb)
    s_ref[...] = _t_dot(a_ref, emb)
    dense_out = jnp.sum(dense * wd_ref[...], axis=0, keepdims=True) + bd_ref[0, 0]
    lin_sum = jnp.sum(lin_ref[...], axis=0, keepdims=True)
    sumsq = jnp.sum(emb * emb, axis=0, keepdims=True)
    aux_ref[...] = dense_out + lin_sum - 0.5 * sumsq


def _stage1b_body(h1p_ref, sp_ref, aux_ref, emb_ref, lin_ref, w1e_ref,
                  b1_ref, a_ref, h1_ref, log_ref, s_ref, q_ref):
    i = pl.program_id(0)
    emb = emb_ref[...]
    h1 = h1p_ref[...] + _t_dot(w1e_ref, emb) + b1_ref[...]
    h1_ref[...] = h1
    s = sp_ref[...] + _t_dot(a_ref, emb)
    fm = 0.5 * (jnp.sum(s * s, axis=0, keepdims=True)
                - jnp.sum(emb * emb, axis=0, keepdims=True))
    lin_sum = jnp.sum(lin_ref[...], axis=0, keepdims=True)
    log_ref[...] = aux_ref[...] + lin_sum + fm

    @pl.when(i == 0)
    def _():
        s_ref[...] = jnp.zeros_like(s_ref)
        q_ref[...] = jnp.zeros_like(q_ref)

    s_ref[...] += jnp.sum(h1, axis=1, keepdims=True)
    q_ref[...] += jnp.sum(h1 * h1, axis=1, keepdims=True)


def _mid_body(h_ref, s_ref, q_ref, g_ref, be_ref, w_ref, b_ref,
              out_ref, s2_ref, q2_ref):
    i = pl.program_id(0)
    mean = s_ref[...] * (1.0 / B)
    var = q_ref[...] * (1.0 / B) - mean * mean
    xh = (h_ref[...] - mean) * lax.rsqrt(var + EPS)
    act = jnp.maximum(g_ref[...] * xh + be_ref[...], 0.0)
    out = _t_dot(w_ref, act) + b_ref[...]
    out_ref[...] = out

    @pl.when(i == 0)
    def _():
        s2_ref[...] = jnp.zeros_like(s2_ref)
        q2_ref[...] = jnp.zeros_like(q2_ref)

    s2_ref[...] += jnp.sum(out, axis=1, keepdims=True)
    q2_ref[...] += jnp.sum(out * out, axis=1, keepdims=True)


def _final_body(h_ref, s_ref, q_ref, g_ref, be_ref, wout_ref, bout_ref,
                log_ref, out_ref):
    mean = s_ref[...] * (1.0 / B)
    var = q_ref[...] * (1.0 / B) - mean * mean
    xh = (h_ref[...] - mean) * lax.rsqrt(var + EPS)
    act = jnp.maximum(g_ref[...] * xh + be_ref[...], 0.0)
    dnn = jnp.sum(act * wout_ref[...], axis=0, keepdims=True) + bout_ref[0, 0]
    out_ref[...] = log_ref[...] + dnn


def _col(r):
    return pl.BlockSpec((r, 1), lambda i: (0, 0))


def _cblk(r):
    return pl.BlockSpec((r, TB), lambda i: (0, i))


def _full(r, c):
    return pl.BlockSpec((r, c), lambda i: (0, 0))


def _dnn_t(dense_t, emb_a, emb_b, lin_a, lin_b, Wd, bd, W1, b1, g1, be1,
           W2, b2, g2, be2, W3, b3, g3, be3, Wout, bout):
    grid = (B // TB,)
    h1_dim, h2_dim, h3_dim = W1.shape[1], W2.shape[1], W3.shape[1]
    kh = FH * D
    fsum_h = jnp.tile(jnp.eye(D, dtype=jnp.float32), (FH, 1))

    h1p, sp, aux = pl.pallas_call(
        _stage1a_body,
        grid=grid,
        in_specs=[
            _cblk(DENSE), _cblk(kh), _cblk(FH),
            _full(DENSE, h1_dim), _full(kh, h1_dim),
            _col(DENSE), _full(1, 1), _full(kh, D),
        ],
        out_specs=[_cblk(h1_dim), _cblk(D), _cblk(1)],
        out_shape=[
            jax.ShapeDtypeStruct((h1_dim, B), jnp.float32),
            jax.ShapeDtypeStruct((D, B), jnp.float32),
            jax.ShapeDtypeStruct((1, B), jnp.float32),
        ],
    )(dense_t, emb_a, lin_a, W1[:DENSE], W1[DENSE:DENSE + kh],
      Wd.reshape(-1, 1), bd.reshape(1, 1), fsum_h)

    h1, logits0, s1, q1 = pl.pallas_call(
        _stage1b_body,
        grid=grid,
        in_specs=[
            _cblk(h1_dim), _cblk(D), _cblk(1), _cblk(kh), _cblk(FH),
            _full(kh, h1_dim), _col(h1_dim), _full(kh, D),
        ],
        out_specs=[_cblk(h1_dim), _cblk(1), _col(h1_dim), _col(h1_dim)],
        out_shape=[
            jax.ShapeDtypeStruct((h1_dim, B), jnp.float32),
            jax.ShapeDtypeStruct((1, B), jnp.float32),
            jax.ShapeDtypeStruct((h1_dim, 1), jnp.float32),
            jax.ShapeDtypeStruct((h1_dim, 1), jnp.float32),
        ],
        input_output_aliases={0: 0},
    )(h1p, sp, aux, emb_b, lin_b, W1[DENSE + kh:], b1.reshape(-1, 1), fsum_h)

    def mid(h, s, q, g, be, w, b, cin, cout):
        return pl.pallas_call(
            _mid_body,
            grid=grid,
            in_specs=[_cblk(cin), _col(cin), _col(cin), _col(cin),
                      _col(cin), _full(cin, cout), _col(cout)],
            out_specs=[_cblk(cout), _col(cout), _col(cout)],
            out_shape=[
                jax.ShapeDtypeStruct((cout, B), jnp.float32),
                jax.ShapeDtypeStruct((cout, 1), jnp.float32),
                jax.ShapeDtypeStruct((cout, 1), jnp.float32),
            ],
        )(h, s, q, g.reshape(-1, 1), be.reshape(-1, 1), w, b.reshape(-1, 1))

    h2, s2, q2 = mid(h1, s1, q1, g1, be1, W2, b2, h1_dim, h2_dim)
    h3, s3, q3 = mid(h2, s2, q2, g2, be2, W3, b3, h2_dim, h3_dim)

    out = pl.pallas_call(
        _final_body,
        grid=grid,
        in_specs=[_cblk(h3_dim), _col(h3_dim), _col(h3_dim), _col(h3_dim),
                  _col(h3_dim), _col(h3_dim), _full(1, 1), _cblk(1)],
        out_specs=_cblk(1),
        out_shape=jax.ShapeDtypeStruct((1, B), jnp.float32),
    )(h3, s3, q3, g3.reshape(-1, 1), be3.reshape(-1, 1),
      Wout.reshape(-1, 1), bout.reshape(1, 1), logits0)
    return out


def kernel(dense_x, discrete_x, linear_tables, embed_tables, Wd, bd,
           W1, b1, g1, be1, W2, b2, g2, be2, W3, b3, g3, be3, Wout, bout):
    idx_t = discrete_x.T
    etab_t = embed_tables.transpose(0, 2, 1)
    gather_a = _make_sc_gather(0)
    gather_b = _make_sc_gather(FH)
    emb_a, lin_a = gather_a(idx_t, etab_t, linear_tables)
    emb_b, lin_b = gather_b(idx_t, etab_t, linear_tables)
    dense_t = dense_x.T
    out_t = _dnn_t(dense_t, emb_a, emb_b, lin_a, lin_b, Wd, bd,
                   W1, b1, g1, be1, W2, b2, g2, be2,
                   W3, b3, g3, be3, Wout, bout)
    return out_t.reshape(B, 1)


# speedup vs baseline: 1.0707x; 1.0707x over previous
import functools

import jax
import jax.numpy as jnp
from jax import lax
from jax.experimental import pallas as pl
from jax.experimental.pallas import tpu as pltpu
from jax.experimental.pallas import tpu_sc as plsc

B = 4096
F = 26
V = 100000
D = 32
DENSE = 13
EPS = 1e-5
TB = 2048
CHUNK = 128
N_CHUNKS = B // CHUNK


def _make_sc_gather():
    info = plsc.get_sparse_core_info()
    nc, ns, nl = info.num_cores, info.num_subcores, info.num_lanes
    nw = nc * ns
    rows_per_w = (F * D) // nw

    mesh = plsc.VectorSubcoreMesh(core_axis_name="c", subcore_axis_name="s")

    @functools.partial(
        pl.kernel,
        out_type=[
            jax.ShapeDtypeStruct((F * D, B), jnp.float32),
            jax.ShapeDtypeStruct((F, B), jnp.float32),
        ],
        mesh=mesh,
        compiler_params=pltpu.CompilerParams(needs_layout_passes=False),
        scratch_types=[
            pltpu.VMEM((V,), jnp.float32),
            pltpu.VMEM((B,), jnp.int32),
            pltpu.VMEM((2, B), jnp.float32),
            pltpu.SemaphoreType.DMA,
        ],
    )
    def sc_gather(idx_hbm, etab_hbm, ltab_hbm, emb_out, lin_out,
                  tab_v, idx_v, out_v, sem_o):
        wid = lax.axis_index("s") * nc + lax.axis_index("c")
        r0 = wid * rows_per_w

        def gather_staged(jb):
            def chunk(k, _):
                base = pl.multiple_of(k * 128, 128)
                for u in range(8):
                    off = base + u * nl
                    iv = idx_v[pl.ds(off, nl)]
                    out_v[jb, pl.ds(off, nl)] = plsc.load_gather(tab_v, [iv])
                return _

            lax.fori_loop(0, B // 128, chunk, None)

        def row_body(j, _):
            r = r0 + j
            f = r // D
            d = r % D

            @pl.when((j == 0) | (d == 0))
            def _():
                pltpu.sync_copy(idx_hbm.at[f], idx_v)

            pltpu.sync_copy(etab_hbm.at[f, d], tab_v)
            jb = j % 2

            @pl.when(j >= 2)
            def _():
                pltpu.make_async_copy(out_v.at[jb], emb_out.at[r], sem_o).wait()

            gather_staged(jb)
            pltpu.make_async_copy(out_v.at[jb], emb_out.at[r], sem_o).start()
            return _

        lax.fori_loop(0, rows_per_w, row_body, None)
        pltpu.make_async_copy(out_v.at[0], emb_out.at[r0], sem_o).wait()
        pltpu.make_async_copy(out_v.at[0], emb_out.at[r0], sem_o).wait()

        @pl.when(wid < F)
        def _():
            pltpu.sync_copy(idx_hbm.at[wid], idx_v)
            pltpu.sync_copy(ltab_hbm.at[wid], tab_v)
            gather_staged(0)
            pltpu.sync_copy(out_v.at[0], lin_out.at[wid])

    return sc_gather


def _t_dot(w_ref, x):
    return lax.dot_general(w_ref[...], x, (((0,), (0,)), ((), ())),
                           preferred_element_type=jnp.float32)


def _stage1_body(dense_ref, emb_ref, lin_ref, w1d_ref, w1e_ref, b1_ref,
                 wd_ref, bd_ref, a_ref, h1_ref, log_ref, s_ref, q_ref):
    i = pl.program_id(0)
    emb = emb_ref[...]
    dense = dense_ref[...]
    h1 = _t_dot(w1d_ref, dense) + _t_dot(w1e_ref, emb) + b1_ref[...]
    h1_ref[...] = h1.astype(h1_ref.dtype)
    s = _t_dot(a_ref, emb)
    fm = 0.5 * (jnp.sum(s * s, axis=0, keepdims=True)
                - jnp.sum(emb * emb, axis=0, keepdims=True))
    dense_out = jnp.sum(dense * wd_ref[...], axis=0, keepdims=True) + bd_ref[0, 0]
    lin_sum = jnp.sum(lin_ref[...], axis=0, keepdims=True)
    log_ref[...] = fm + dense_out + lin_sum

    @pl.when(i == 0)
    def _():
        s_ref[...] = jnp.zeros_like(s_ref)
        q_ref[...] = jnp.zeros_like(q_ref)

    s_ref[...] += jnp.sum(h1, axis=1, keepdims=True)
    q_ref[...] += jnp.sum(h1 * h1, axis=1, keepdims=True)


def _mid_body(h_ref, s_ref, q_ref, g_ref, be_ref, w_ref, b_ref,
              out_ref, s2_ref, q2_ref):
    i = pl.program_id(0)
    mean = s_ref[...] * (1.0 / B)
    var = q_ref[...] * (1.0 / B) - mean * mean
    xh = (h_ref[...].astype(jnp.float32) - mean) * lax.rsqrt(var + EPS)
    act = jnp.maximum(g_ref[...] * xh + be_ref[...], 0.0)
    out = _t_dot(w_ref, act) + b_ref[...]
    out_ref[...] = out.astype(out_ref.dtype)

    @pl.when(i == 0)
    def _():
        s2_ref[...] = jnp.zeros_like(s2_ref)
        q2_ref[...] = jnp.zeros_like(q2_ref)

    s2_ref[...] += jnp.sum(out, axis=1, keepdims=True)
    q2_ref[...] += jnp.sum(out * out, axis=1, keepdims=True)


def _final_body(h_ref, s_ref, q_ref, g_ref, be_ref, wout_ref, bout_ref,
                log_ref, out_ref):
    mean = s_ref[...] * (1.0 / B)
    var = q_ref[...] * (1.0 / B) - mean * mean
    xh = (h_ref[...].astype(jnp.float32) - mean) * lax.rsqrt(var + EPS)
    act = jnp.maximum(g_ref[...] * xh + be_ref[...], 0.0)
    dnn = jnp.sum(act * wout_ref[...], axis=0, keepdims=True) + bout_ref[0, 0]
    out_ref[...] = log_ref[...] + dnn


def _col(r):
    return pl.BlockSpec((r, 1), lambda i: (0, 0))


def _cblk(r):
    return pl.BlockSpec((r, TB), lambda i: (0, i))


def _full(r, c):
    return pl.BlockSpec((r, c), lambda i: (0, 0))


def _dnn_t(dense_t, emb_t, lin_t, Wd, bd, W1, b1, g1, be1, W2, b2, g2, be2,
           W3, b3, g3, be3, Wout, bout):
    grid = (B // TB,)
    h1_dim, h2_dim, h3_dim = W1.shape[1], W2.shape[1], W3.shape[1]
    fsum = jnp.tile(jnp.eye(D, dtype=jnp.float32), (F, 1))

    h1, logits0, s1, q1 = pl.pallas_call(
        _stage1_body,
        grid=grid,
        in_specs=[
            _cblk(DENSE), _cblk(F * D), _cblk(F),
            _full(DENSE, h1_dim), _full(F * D, h1_dim), _col(h1_dim),
            _col(DENSE), _full(1, 1), _full(F * D, D),
        ],
        out_specs=[_cblk(h1_dim), _cblk(1), _col(h1_dim), _col(h1_dim)],
        out_shape=[
            jax.ShapeDtypeStruct((h1_dim, B), jnp.bfloat16),
            jax.ShapeDtypeStruct((1, B), jnp.float32),
            jax.ShapeDtypeStruct((h1_dim, 1), jnp.float32),
            jax.ShapeDtypeStruct((h1_dim, 1), jnp.float32),
        ],
    )(dense_t, emb_t, lin_t, W1[:DENSE], W1[DENSE:], b1.reshape(-1, 1),
      Wd.reshape(-1, 1), bd.reshape(1, 1), fsum)

    def mid(h, s, q, g, be, w, b, cin, cout):
        return pl.pallas_call(
            _mid_body,
            grid=grid,
            in_specs=[_cblk(cin), _col(cin), _col(cin), _col(cin),
                      _col(cin), _full(cin, cout), _col(cout)],
            out_specs=[_cblk(cout), _col(cout), _col(cout)],
            out_shape=[
                jax.ShapeDtypeStruct((cout, B), jnp.bfloat16),
                jax.ShapeDtypeStruct((cout, 1), jnp.float32),
                jax.ShapeDtypeStruct((cout, 1), jnp.float32),
            ],
        )(h, s, q, g.reshape(-1, 1), be.reshape(-1, 1), w, b.reshape(-1, 1))

    h2, s2, q2 = mid(h1, s1, q1, g1, be1, W2, b2, h1_dim, h2_dim)
    h3, s3, q3 = mid(h2, s2, q2, g2, be2, W3, b3, h2_dim, h3_dim)

    out = pl.pallas_call(
        _final_body,
        grid=grid,
        in_specs=[_cblk(h3_dim), _col(h3_dim), _col(h3_dim), _col(h3_dim),
                  _col(h3_dim), _col(h3_dim), _full(1, 1), _cblk(1)],
        out_specs=_cblk(1),
        out_shape=jax.ShapeDtypeStruct((1, B), jnp.float32),
    )(h3, s3, q3, g3.reshape(-1, 1), be3.reshape(-1, 1),
      Wout.reshape(-1, 1), bout.reshape(1, 1), logits0)
    return out


def kernel(dense_x, discrete_x, linear_tables, embed_tables, Wd, bd,
           W1, b1, g1, be1, W2, b2, g2, be2, W3, b3, g3, be3, Wout, bout):
    sc_gather = _make_sc_gather()
    idx_t = discrete_x.T
    etab_t = embed_tables.transpose(0, 2, 1)
    emb_t, lin_t = sc_gather(idx_t, etab_t, linear_tables)
    dense_t = dense_x.T
    out_t = _dnn_t(dense_t, emb_t, lin_t, Wd, bd, W1, b1, g1, be1,
                   W2, b2, g2, be2, W3, b3, g3, be3, Wout, bout)
    return out_t.reshape(B, 1)
